# baseline (speedup 1.0000x reference)
import jax
import jax.numpy as jnp
from jax.experimental import pallas as pl
from jax.experimental.pallas import tpu as pltpu

E = 16
K = 2
D = 1024
F = 512
T = 1024
B = 128
EH = E // 2


def _moe_body(x_ref, gate_ref, w13a_ref, w2a_ref, w13b_ref, w2b_ref,
              out_ref, rank_ref, comb_ref):
    p = pl.program_id(0)

    @pl.when(p == 0)
    def _routing():
        x = x_ref[...]
        logits = jax.lax.dot_general(
            gate_ref[...], x, (((1,), (1,)), ((), ())),
            preferred_element_type=jnp.float32)
        ii = jax.lax.broadcasted_iota(jnp.int32, (E, T), 0)
        m1 = jnp.max(logits, axis=0, keepdims=True)
        i1 = jnp.min(jnp.where(logits == m1, ii, E), axis=0, keepdims=True)
        masked = jnp.where(ii == i1, -jnp.inf, logits)
        m2 = jnp.max(masked, axis=0, keepdims=True)
        i2 = jnp.min(jnp.where(masked == m2, ii, E), axis=0, keepdims=True)
        dd = jnp.exp(m2 - m1)
        w1 = 1.0 / (1.0 + dd)
        w2 = dd / (1.0 + dd)
        sel1 = ii == i1
        sel2 = ii == i2
        comb_ref[...] = jnp.where(sel1, w1, 0.0) + jnp.where(sel2, w2, 0.0)
        mask = (sel1 | sel2).astype(jnp.float32)
        ta = jax.lax.broadcasted_iota(jnp.int32, (T, T), 0)
        tb = jax.lax.broadcasted_iota(jnp.int32, (T, T), 1)
        lt = (ta < tb).astype(jnp.float32)
        rank = jax.lax.dot_general(mask, lt, (((1,), (0,)), ((), ())),
                                   preferred_element_type=jnp.float32)
        rank_ref[...] = jnp.where(mask > 0.0, rank, 2.0 * T)
        out_ref[...] = jnp.zeros_like(out_ref)

    def _count(e):
        rr = rank_ref[pl.ds(e, 1), :]
        return jnp.sum(jnp.where(rr < 2.0 * T, 1.0, 0.0)).astype(jnp.int32)

    na = (_count(p) + (B - 1)) // B
    nbk = (_count(p + EH) + (B - 1)) // B
    nb = jnp.maximum(na, nbk)

    def _chain(j, e, w13_ref, w2_ref):
        svec = (j * B + jax.lax.broadcasted_iota(jnp.int32, (B, 1), 0)
                ).astype(jnp.float32)
        G = (rank_ref[pl.ds(e, 1), :] == svec).astype(jnp.float32)
        rows = jax.lax.dot_general(G, x_ref[...], (((1,), (0,)), ((), ())),
                                   preferred_element_type=jnp.float32)
        g = jax.lax.dot_general(rows, w13_ref[0, :F, :], (((1,), (1,)), ((), ())),
                                preferred_element_type=jnp.float32)
        u = jax.lax.dot_general(rows, w13_ref[0, F:, :], (((1,), (1,)), ((), ())),
                                preferred_element_type=jnp.float32)
        act = g / (1.0 + jnp.exp(-g)) * u
        o = jax.lax.dot_general(act, w2_ref[0], (((1,), (1,)), ((), ())),
                                preferred_element_type=jnp.float32)
        GW = G * comb_ref[pl.ds(e, 1), :]
        return jax.lax.dot_general(GW, o, (((0,), (0,)), ((), ())),
                                   preferred_element_type=jnp.float32)

    def _block(j, _):
        ca = _chain(j, p, w13a_ref, w2a_ref)
        cb = _chain(j, p + EH, w13b_ref, w2b_ref)
        out_ref[...] += ca + cb
        return _

    jax.lax.fori_loop(0, nb, _block, None)


@jax.jit
def kernel(hidden_states, gate_weight, w13_weight, w2_weight):
    return pl.pallas_call(
        _moe_body,
        grid=(EH,),
        in_specs=[
            pl.BlockSpec((T, D), lambda p: (0, 0)),
            pl.BlockSpec((E, D), lambda p: (0, 0)),
            pl.BlockSpec((1, 2 * F, D), lambda p: (p, 0, 0)),
            pl.BlockSpec((1, D, F), lambda p: (p, 0, 0)),
            pl.BlockSpec((1, 2 * F, D), lambda p: (p + EH, 0, 0)),
            pl.BlockSpec((1, D, F), lambda p: (p + EH, 0, 0)),
        ],
        out_specs=pl.BlockSpec((T, D), lambda p: (0, 0)),
        out_shape=jax.ShapeDtypeStruct((T, D), jnp.float32),
        scratch_shapes=[
            pltpu.VMEM((E, T), jnp.float32),
            pltpu.VMEM((E, T), jnp.float32),
        ],
        compiler_params=pltpu.CompilerParams(
            dimension_semantics=("arbitrary",),
        ),
    )(hidden_states, gate_weight, w13_weight, w2_weight,
      w13_weight, w2_weight)


# speedup vs baseline: 1.0821x; 1.0821x over previous
import jax
import jax.numpy as jnp
from jax.experimental import pallas as pl
from jax.experimental.pallas import tpu as pltpu

E = 16
K = 2
D = 1024
F = 512
T = 1024
B = 128
EH = E // 2


def _moe_body(x_ref, gate_ref, w13_ref, w2_ref, out_ref, rank_ref, comb_ref):
    p = pl.program_id(0)

    @pl.when(p == 0)
    def _routing():
        x = x_ref[...]
        logits = jax.lax.dot_general(
            gate_ref[...], x, (((1,), (1,)), ((), ())),
            preferred_element_type=jnp.float32)
        ii = jax.lax.broadcasted_iota(jnp.int32, (E, T), 0)
        m1 = jnp.max(logits, axis=0, keepdims=True)
        i1 = jnp.min(jnp.where(logits == m1, ii, E), axis=0, keepdims=True)
        masked = jnp.where(ii == i1, -jnp.inf, logits)
        m2 = jnp.max(masked, axis=0, keepdims=True)
        i2 = jnp.min(jnp.where(masked == m2, ii, E), axis=0, keepdims=True)
        dd = jnp.exp(m2 - m1)
        w1 = 1.0 / (1.0 + dd)
        w2 = dd / (1.0 + dd)
        sel1 = ii == i1
        sel2 = ii == i2
        comb_ref[...] = jnp.where(sel1, w1, 0.0) + jnp.where(sel2, w2, 0.0)
        mask = (sel1 | sel2).astype(jnp.float32)
        ta = jax.lax.broadcasted_iota(jnp.int32, (T, T), 0)
        tb = jax.lax.broadcasted_iota(jnp.int32, (T, T), 1)
        lt = (ta < tb).astype(jnp.float32)
        rank = jax.lax.dot_general(mask, lt, (((1,), (0,)), ((), ())),
                                   preferred_element_type=jnp.float32)
        rank_ref[...] = jnp.where(mask > 0.0, rank, 2.0 * T)
        out_ref[...] = jnp.zeros_like(out_ref)

    count = jnp.sum(jnp.where(rank_ref[pl.ds(p, 1), :] < 2.0 * T, 1.0, 0.0)
                    ).astype(jnp.int32)
    nb = (count + (B - 1)) // B

    def _gather(j):
        svec = (j * B + jax.lax.broadcasted_iota(jnp.int32, (B, 1), 0)
                ).astype(jnp.float32)
        G = (rank_ref[pl.ds(p, 1), :] == svec).astype(jnp.float32)
        rows = jax.lax.dot_general(G, x_ref[...], (((1,), (0,)), ((), ())),
                                   preferred_element_type=jnp.float32)
        GW = G * comb_ref[pl.ds(p, 1), :]
        return rows, GW

    def _ffn_scatter(rows, GW):
        g = jax.lax.dot_general(rows, w13_ref[0, :F, :], (((1,), (1,)), ((), ())),
                                preferred_element_type=jnp.float32)
        u = jax.lax.dot_general(rows, w13_ref[0, F:, :], (((1,), (1,)), ((), ())),
                                preferred_element_type=jnp.float32)
        act = g / (1.0 + jnp.exp(-g)) * u
        o = jax.lax.dot_general(act, w2_ref[0], (((1,), (1,)), ((), ())),
                                preferred_element_type=jnp.float32)
        out_ref[...] += jax.lax.dot_general(
            GW, o, (((0,), (0,)), ((), ())),
            preferred_element_type=jnp.float32)

    def _block(j, carry):
        rows_c, gw_c = carry
        _ffn_scatter(rows_c, gw_c)
        return _gather(j)

    carry = _gather(0)
    carry = jax.lax.fori_loop(1, nb, _block, carry)
    _ffn_scatter(*carry)


@jax.jit
def kernel(hidden_states, gate_weight, w13_weight, w2_weight):
    return pl.pallas_call(
        _moe_body,
        grid=(E,),
        in_specs=[
            pl.BlockSpec((T, D), lambda p: (0, 0)),
            pl.BlockSpec((E, D), lambda p: (0, 0)),
            pl.BlockSpec((1, 2 * F, D), lambda p: (p, 0, 0)),
            pl.BlockSpec((1, D, F), lambda p: (p, 0, 0)),
        ],
        out_specs=pl.BlockSpec((T, D), lambda p: (0, 0)),
        out_shape=jax.ShapeDtypeStruct((T, D), jnp.float32),
        scratch_shapes=[
            pltpu.VMEM((E, T), jnp.float32),
            pltpu.VMEM((E, T), jnp.float32),
        ],
        compiler_params=pltpu.CompilerParams(
            dimension_semantics=("arbitrary",),
        ),
    )(hidden_states, gate_weight, w13_weight, w2_weight)


# speedup vs baseline: 1.0887x; 1.0062x over previous
import jax
import jax.numpy as jnp
from jax.experimental import pallas as pl
from jax.experimental.pallas import tpu as pltpu

E = 16
K = 2
D = 1024
F = 512
T = 1024
B = 128
EH = E // 2


def _moe_body(x_ref, gate_ref, w13_ref, w2_ref, out_ref, rank_ref, comb_ref,
              xbf_ref):
    p = pl.program_id(0)

    @pl.when(p == 0)
    def _routing():
        x = x_ref[...]
        logits = jax.lax.dot_general(
            gate_ref[...], x, (((1,), (1,)), ((), ())),
            preferred_element_type=jnp.float32)
        ii = jax.lax.broadcasted_iota(jnp.int32, (E, T), 0)
        m1 = jnp.max(logits, axis=0, keepdims=True)
        i1 = jnp.min(jnp.where(logits == m1, ii, E), axis=0, keepdims=True)
        masked = jnp.where(ii == i1, -jnp.inf, logits)
        m2 = jnp.max(masked, axis=0, keepdims=True)
        i2 = jnp.min(jnp.where(masked == m2, ii, E), axis=0, keepdims=True)
        dd = jnp.exp(m2 - m1)
        w1 = 1.0 / (1.0 + dd)
        w2 = dd / (1.0 + dd)
        sel1 = ii == i1
        sel2 = ii == i2
        comb_ref[...] = jnp.where(sel1, w1, 0.0) + jnp.where(sel2, w2, 0.0)
        mask = (sel1 | sel2).astype(jnp.float32)
        ta = jax.lax.broadcasted_iota(jnp.int32, (T, T), 0)
        tb = jax.lax.broadcasted_iota(jnp.int32, (T, T), 1)
        lt = (ta < tb).astype(jnp.float32)
        rank = jax.lax.dot_general(mask, lt, (((1,), (0,)), ((), ())),
                                   preferred_element_type=jnp.float32)
        rank_ref[...] = jnp.where(mask > 0.0, rank, 2.0 * T)
        out_ref[...] = jnp.zeros_like(out_ref)
        xbf_ref[...] = x.astype(jnp.bfloat16)

    count = jnp.sum(jnp.where(rank_ref[pl.ds(p, 1), :] < 2.0 * T, 1.0, 0.0)
                    ).astype(jnp.int32)
    nb = (count + (B - 1)) // B

    def _gather(j):
        svec = (j * B + jax.lax.broadcasted_iota(jnp.int32, (B, 1), 0)
                ).astype(jnp.float32)
        G = (rank_ref[pl.ds(p, 1), :] == svec)
        rows = jax.lax.dot_general(
            G.astype(jnp.bfloat16), xbf_ref[...], (((1,), (0,)), ((), ())),
            preferred_element_type=jnp.float32)
        GW = (G.astype(jnp.float32) * comb_ref[pl.ds(p, 1), :]
              ).astype(jnp.bfloat16)
        return rows, GW

    def _ffn_scatter(rows, GW):
        g = jax.lax.dot_general(rows, w13_ref[0, :F, :], (((1,), (1,)), ((), ())),
                                preferred_element_type=jnp.float32)
        u = jax.lax.dot_general(rows, w13_ref[0, F:, :], (((1,), (1,)), ((), ())),
                                preferred_element_type=jnp.float32)
        act = g / (1.0 + jnp.exp(-g)) * u
        o = jax.lax.dot_general(act, w2_ref[0], (((1,), (1,)), ((), ())),
                                preferred_element_type=jnp.float32)
        out_ref[...] += jax.lax.dot_general(
            GW, o.astype(jnp.bfloat16), (((0,), (0,)), ((), ())),
            preferred_element_type=jnp.float32)

    def _block(j, carry):
        rows_c, gw_c = carry
        _ffn_scatter(rows_c, gw_c)
        return _gather(j)

    carry = _gather(0)
    carry = jax.lax.fori_loop(1, nb, _block, carry)
    _ffn_scatter(*carry)


@jax.jit
def kernel(hidden_states, gate_weight, w13_weight, w2_weight):
    return pl.pallas_call(
        _moe_body,
        grid=(E,),
        in_specs=[
            pl.BlockSpec((T, D), lambda p: (0, 0)),
            pl.BlockSpec((E, D), lambda p: (0, 0)),
            pl.BlockSpec((1, 2 * F, D), lambda p: (p, 0, 0)),
            pl.BlockSpec((1, D, F), lambda p: (p, 0, 0)),
        ],
        out_specs=pl.BlockSpec((T, D), lambda p: (0, 0)),
        out_shape=jax.ShapeDtypeStruct((T, D), jnp.float32),
        scratch_shapes=[
            pltpu.VMEM((E, T), jnp.float32),
            pltpu.VMEM((E, T), jnp.float32),
            pltpu.VMEM((T, D), jnp.bfloat16),
        ],
        compiler_params=pltpu.CompilerParams(
            dimension_semantics=("arbitrary",),
        ),
    )(hidden_states, gate_weight, w13_weight, w2_weight)


# speedup vs baseline: 1.4240x; 1.3079x over previous
import jax
import jax.numpy as jnp
from jax.experimental import pallas as pl
from jax.experimental.pallas import tpu as pltpu

E = 16
K = 2
D = 1024
F = 512
T = 1024
B = 256
EH = E // 2


def _moe_body(x_ref, gate_ref, w13_ref, w2_ref, out_ref, rank_ref, comb_ref):
    p = pl.program_id(0)

    @pl.when(p == 0)
    def _routing():
        x = x_ref[...]
        logits = jax.lax.dot_general(
            gate_ref[...], x, (((1,), (1,)), ((), ())),
            preferred_element_type=jnp.float32)
        ii = jax.lax.broadcasted_iota(jnp.int32, (E, T), 0)
        m1 = jnp.max(logits, axis=0, keepdims=True)
        i1 = jnp.min(jnp.where(logits == m1, ii, E), axis=0, keepdims=True)
        masked = jnp.where(ii == i1, -jnp.inf, logits)
        m2 = jnp.max(masked, axis=0, keepdims=True)
        i2 = jnp.min(jnp.where(masked == m2, ii, E), axis=0, keepdims=True)
        dd = jnp.exp(m2 - m1)
        w1 = 1.0 / (1.0 + dd)
        w2 = dd / (1.0 + dd)
        sel1 = ii == i1
        sel2 = ii == i2
        comb_ref[...] = jnp.where(sel1, w1, 0.0) + jnp.where(sel2, w2, 0.0)
        mask = (sel1 | sel2).astype(jnp.float32)
        ta = jax.lax.broadcasted_iota(jnp.int32, (T, T), 0)
        tb = jax.lax.broadcasted_iota(jnp.int32, (T, T), 1)
        lt = (ta < tb).astype(jnp.float32)
        rank = jax.lax.dot_general(mask, lt, (((1,), (0,)), ((), ())),
                                   preferred_element_type=jnp.float32)
        rank_ref[...] = jnp.where(mask > 0.0, rank, 2.0 * T)
        out_ref[...] = jnp.zeros_like(out_ref)

    count = jnp.sum(jnp.where(rank_ref[pl.ds(p, 1), :] < 2.0 * T, 1.0, 0.0)
                    ).astype(jnp.int32)
    nb = (count + (B - 1)) // B

    def _gather(j):
        svec = (j * B + jax.lax.broadcasted_iota(jnp.int32, (B, 1), 0)
                ).astype(jnp.float32)
        G = (rank_ref[pl.ds(p, 1), :] == svec).astype(jnp.float32)
        rows = jax.lax.dot_general(G, x_ref[...], (((1,), (0,)), ((), ())),
                                   preferred_element_type=jnp.float32)
        GW = G * comb_ref[pl.ds(p, 1), :]
        return rows, GW

    def _ffn_scatter(rows, GW):
        g = jax.lax.dot_general(rows, w13_ref[0, :F, :], (((1,), (1,)), ((), ())),
                                preferred_element_type=jnp.float32)
        u = jax.lax.dot_general(rows, w13_ref[0, F:, :], (((1,), (1,)), ((), ())),
                                preferred_element_type=jnp.float32)
        act = g / (1.0 + jnp.exp(-g)) * u
        o = jax.lax.dot_general(act, w2_ref[0], (((1,), (1,)), ((), ())),
                                preferred_element_type=jnp.float32)
        out_ref[...] += jax.lax.dot_general(
            GW, o, (((0,), (0,)), ((), ())),
            preferred_element_type=jnp.float32)

    def _block(j, carry):
        rows_c, gw_c = carry
        _ffn_scatter(rows_c, gw_c)
        return _gather(j)

    carry = _gather(0)
    carry = jax.lax.fori_loop(1, nb, _block, carry)
    _ffn_scatter(*carry)


@jax.jit
def kernel(hidden_states, gate_weight, w13_weight, w2_weight):
    return pl.pallas_call(
        _moe_body,
        grid=(E,),
        in_specs=[
            pl.BlockSpec((T, D), lambda p: (0, 0)),
            pl.BlockSpec((E, D), lambda p: (0, 0)),
            pl.BlockSpec((1, 2 * F, D), lambda p: (p, 0, 0)),
            pl.BlockSpec((1, D, F), lambda p: (p, 0, 0)),
        ],
        out_specs=pl.BlockSpec((T, D), lambda p: (0, 0)),
        out_shape=jax.ShapeDtypeStruct((T, D), jnp.float32),
        scratch_shapes=[
            pltpu.VMEM((E, T), jnp.float32),
            pltpu.VMEM((E, T), jnp.float32),
        ],
        compiler_params=pltpu.CompilerParams(
            dimension_semantics=("arbitrary",),
        ),
    )(hidden_states, gate_weight, w13_weight, w2_weight)


# speedup vs baseline: 1.4277x; 1.0026x over previous
import jax
import jax.numpy as jnp
from jax.experimental import pallas as pl
from jax.experimental.pallas import tpu as pltpu

E = 16
K = 2
D = 1024
F = 512
T = 1024
B = 192
EH = E // 2


def _moe_body(x_ref, gate_ref, w13_ref, w2_ref, out_ref, rank_ref, comb_ref):
    p = pl.program_id(0)

    @pl.when(p == 0)
    def _routing():
        x = x_ref[...]
        logits = jax.lax.dot_general(
            gate_ref[...], x, (((1,), (1,)), ((), ())),
            preferred_element_type=jnp.float32)
        ii = jax.lax.broadcasted_iota(jnp.int32, (E, T), 0)
        m1 = jnp.max(logits, axis=0, keepdims=True)
        i1 = jnp.min(jnp.where(logits == m1, ii, E), axis=0, keepdims=True)
        masked = jnp.where(ii == i1, -jnp.inf, logits)
        m2 = jnp.max(masked, axis=0, keepdims=True)
        i2 = jnp.min(jnp.where(masked == m2, ii, E), axis=0, keepdims=True)
        dd = jnp.exp(m2 - m1)
        w1 = 1.0 / (1.0 + dd)
        w2 = dd / (1.0 + dd)
        sel1 = ii == i1
        sel2 = ii == i2
        comb_ref[...] = jnp.where(sel1, w1, 0.0) + jnp.where(sel2, w2, 0.0)
        mask = (sel1 | sel2).astype(jnp.float32)
        ta = jax.lax.broadcasted_iota(jnp.int32, (T, T), 0)
        tb = jax.lax.broadcasted_iota(jnp.int32, (T, T), 1)
        lt = (ta < tb).astype(jnp.float32)
        rank = jax.lax.dot_general(mask, lt, (((1,), (0,)), ((), ())),
                                   preferred_element_type=jnp.float32)
        rank_ref[...] = jnp.where(mask > 0.0, rank, 2.0 * T)
        out_ref[...] = jnp.zeros_like(out_ref)

    count = jnp.sum(jnp.where(rank_ref[pl.ds(p, 1), :] < 2.0 * T, 1.0, 0.0)
                    ).astype(jnp.int32)
    nb = (count + (B - 1)) // B

    def _gather(j):
        svec = (j * B + jax.lax.broadcasted_iota(jnp.int32, (B, 1), 0)
                ).astype(jnp.float32)
        G = (rank_ref[pl.ds(p, 1), :] == svec).astype(jnp.float32)
        rows = jax.lax.dot_general(G, x_ref[...], (((1,), (0,)), ((), ())),
                                   preferred_element_type=jnp.float32)
        GW = G * comb_ref[pl.ds(p, 1), :]
        return rows, GW

    def _ffn_scatter(rows, GW):
        g = jax.lax.dot_general(rows, w13_ref[0, :F, :], (((1,), (1,)), ((), ())),
                                preferred_element_type=jnp.float32)
        u = jax.lax.dot_general(rows, w13_ref[0, F:, :], (((1,), (1,)), ((), ())),
                                preferred_element_type=jnp.float32)
        act = g / (1.0 + jnp.exp(-g)) * u
        o = jax.lax.dot_general(act, w2_ref[0], (((1,), (1,)), ((), ())),
                                preferred_element_type=jnp.float32)
        out_ref[...] += jax.lax.dot_general(
            GW, o, (((0,), (0,)), ((), ())),
            preferred_element_type=jnp.float32)

    def _block(j, carry):
        rows_c, gw_c = carry
        _ffn_scatter(rows_c, gw_c)
        return _gather(j)

    carry = _gather(0)
    carry = jax.lax.fori_loop(1, nb, _block, carry)
    _ffn_scatter(*carry)


@jax.jit
def kernel(hidden_states, gate_weight, w13_weight, w2_weight):
    return pl.pallas_call(
        _moe_body,
        grid=(E,),
        in_specs=[
            pl.BlockSpec((T, D), lambda p: (0, 0)),
            pl.BlockSpec((E, D), lambda p: (0, 0)),
            pl.BlockSpec((1, 2 * F, D), lambda p: (p, 0, 0)),
            pl.BlockSpec((1, D, F), lambda p: (p, 0, 0)),
        ],
        out_specs=pl.BlockSpec((T, D), lambda p: (0, 0)),
        out_shape=jax.ShapeDtypeStruct((T, D), jnp.float32),
        scratch_shapes=[
            pltpu.VMEM((E, T), jnp.float32),
            pltpu.VMEM((E, T), jnp.float32),
        ],
        compiler_params=pltpu.CompilerParams(
            dimension_semantics=("arbitrary",),
        ),
    )(hidden_states, gate_weight, w13_weight, w2_weight)
